# Initial kernel scaffold; baseline (speedup 1.0000x reference)
import jax
import jax.numpy as jnp
from jax.experimental import pallas as pl


def kernel(x, x_img, q, embedding, W_linear, W_rule):
    raise NotImplementedError("write your pallas kernel here")


# speedup vs baseline: 17.2208x; 17.2208x over previous
import functools

import jax
import jax.numpy as jnp
from jax.experimental import pallas as pl

BSZ = 1024
L = 64
D = 32
NC = 10
LVLS = 6
OPAD = 16

_HI = jax.lax.Precision.HIGHEST


def _rms(x):
    return x / (jnp.sqrt(jnp.mean(x * x, axis=1, keepdims=True) + 1e-06) + 1e-06)


def _combine_level(a1, a2, wstack):
    n = a1.shape[0]
    ts = jnp.stack(
        [
            jnp.dot(wstack, a2[j], preferred_element_type=jnp.float32, precision=_HI)
            for j in range(n)
        ],
        axis=0,
    )
    s0 = jnp.sum(a1 * ts[:, :D], axis=1, keepdims=True)
    s1 = jnp.sum(a1 * ts[:, D:], axis=1, keepdims=True)
    m = jnp.maximum(s0, s1)
    e0 = jnp.exp(s0 - m)
    e1 = jnp.exp(s1 - m)
    s = e0 + e1
    return _rms((e0 / s) * a1 + (e1 / s) * a2)


def _tree_body(x3_ref, q3_ref, embT_ref, wstack_ref, wl_ref, out_ref, *, B):
    x3 = x3_ref[...]
    embT = embT_ref[...]
    wstack = wstack_ref[...]
    wl = wl_ref[...]
    ql = q3_ref[0:1]
    qh = q3_ref[1:2]

    h = jnp.zeros((L, D, B), jnp.float32)
    for c in range(NC + 1):
        row = jnp.broadcast_to(embT[:, c : c + 1], (D, B))[None]
        h = jnp.where(x3 == c, row, h)
    h = _rms(h)

    levels = [h]
    for _ in range(LVLS):
        cur = levels[-1]
        n = cur.shape[0]
        pairs = cur.reshape(n // 2, 2, D, B)
        levels.append(_combine_level(pairs[:, 0], pairs[:, 1], wstack))
    levels = levels[::-1]

    inf3 = jnp.broadcast_to(embT[:, NC : NC + 1], (D, B))[None]

    idx = jax.lax.broadcasted_iota(jnp.int32, (L, 1, B), 0)
    full = jnp.logical_and(ql <= idx, qh >= idx)
    res = jnp.where(full, levels[LVLS], inf3)
    n, w = L, 1
    for d in range(LVLS - 1, -1, -1):
        n //= 2
        w *= 2
        pairs = res.reshape(n, 2, D, B)
        comb = _combine_level(pairs[:, 0], pairs[:, 1], wstack)
        j = jax.lax.broadcasted_iota(jnp.int32, (n, 1, B), 0)
        lo = j * w
        hi = lo + (w - 1)
        fullm = jnp.logical_and(ql <= lo, qh >= hi)
        nonem = jnp.logical_or(ql > hi, qh < lo)
        res = jnp.where(fullm, levels[d], jnp.where(nonem, inf3, comb))

    root = res[0]
    out_ref[...] = jnp.dot(wl, root, preferred_element_type=jnp.float32, precision=_HI)


def _run(x3, q3, embT, wstack, wlp, *, B, interpret=False):
    grid = (BSZ // B,)
    return pl.pallas_call(
        functools.partial(_tree_body, B=B),
        grid=grid,
        in_specs=[
            pl.BlockSpec((L, 1, B), lambda i: (0, 0, i)),
            pl.BlockSpec((2, 1, B), lambda i: (0, 0, i)),
            pl.BlockSpec((D, 16), lambda i: (0, 0)),
            pl.BlockSpec((2 * D, D), lambda i: (0, 0)),
            pl.BlockSpec((OPAD, D), lambda i: (0, 0)),
        ],
        out_specs=pl.BlockSpec((OPAD, B), lambda i: (0, i)),
        out_shape=jax.ShapeDtypeStruct((OPAD, BSZ), jnp.float32),
        interpret=interpret,
    )(x3, q3, embT, wstack, wlp)


@jax.jit
def kernel(x, x_img, q, embedding, W_linear, W_rule):
    del x_img
    B = 256
    x3 = x.astype(jnp.int32).T[:, None, :]
    q3 = q.astype(jnp.int32).T[:, None, :]
    embT = jnp.zeros((D, 16), jnp.float32).at[:, : NC + 1].set(embedding.T)
    wstack = W_rule.reshape(2 * D, D)
    wlp = jnp.zeros((OPAD, D), jnp.float32).at[:NC].set(W_linear)
    out = _run(x3, q3, embT, wstack, wlp, B=B)
    return out.T[:, :NC]


# speedup vs baseline: 18.3986x; 1.0684x over previous
import functools

import jax
import jax.numpy as jnp
from jax.experimental import pallas as pl

BSZ = 1024
L = 64
D = 32
NC = 10
LVLS = 6
OPAD = 16

_HI = jax.lax.Precision.HIGHEST


def _rms(x):
    return x / (jnp.sqrt(jnp.mean(x * x, axis=1, keepdims=True) + 1e-06) + 1e-06)


def _combine_level(a1, a2, wstack):
    n = a1.shape[0]
    ts = jnp.stack(
        [
            jnp.dot(wstack, a2[j], preferred_element_type=jnp.float32, precision=_HI)
            for j in range(n)
        ],
        axis=0,
    )
    s0 = jnp.sum(a1 * ts[:, :D], axis=1, keepdims=True)
    s1 = jnp.sum(a1 * ts[:, D:], axis=1, keepdims=True)
    m = jnp.maximum(s0, s1)
    e0 = jnp.exp(s0 - m)
    e1 = jnp.exp(s1 - m)
    s = e0 + e1
    return _rms((e0 / s) * a1 + (e1 / s) * a2)


def _tree_body(x3_ref, q3_ref, embT_ref, wstack_ref, wl_ref, out_ref, *, B):
    x3 = x3_ref[...]
    embT = embT_ref[...]
    wstack = wstack_ref[...]
    wl = wl_ref[...]
    ql = q3_ref[0:1]
    qh = q3_ref[1:2]

    h = jnp.zeros((L, D, B), jnp.float32)
    for c in range(NC + 1):
        row = jnp.broadcast_to(embT[:, c : c + 1], (D, B))[None]
        h = jnp.where(x3 == c, row, h)
    h = _rms(h)

    levels = [h]
    for _ in range(LVLS):
        cur = levels[-1]
        n = cur.shape[0]
        pairs = cur.reshape(n // 2, 2, D, B)
        levels.append(_combine_level(pairs[:, 0], pairs[:, 1], wstack))
    levels = levels[::-1]

    inf3 = jnp.broadcast_to(embT[:, NC : NC + 1], (D, B))[None]

    idx = jax.lax.broadcasted_iota(jnp.int32, (L, 1, B), 0)
    full = jnp.logical_and(ql <= idx, qh >= idx)
    res = jnp.where(full, levels[LVLS], inf3)
    n, w = L, 1
    for d in range(LVLS - 1, -1, -1):
        n //= 2
        w *= 2
        pairs = res.reshape(n, 2, D, B)
        comb = _combine_level(pairs[:, 0], pairs[:, 1], wstack)
        j = jax.lax.broadcasted_iota(jnp.int32, (n, 1, B), 0)
        lo = j * w
        hi = lo + (w - 1)
        fullm = jnp.logical_and(ql <= lo, qh >= hi)
        nonem = jnp.logical_or(ql > hi, qh < lo)
        res = jnp.where(fullm, levels[d], jnp.where(nonem, inf3, comb))

    root = res[0]
    out_ref[...] = jnp.dot(wl, root, preferred_element_type=jnp.float32, precision=_HI)


def _run(x3, q3, embT, wstack, wlp, *, B, interpret=False):
    grid = (BSZ // B,)
    return pl.pallas_call(
        functools.partial(_tree_body, B=B),
        grid=grid,
        in_specs=[
            pl.BlockSpec((L, 1, B), lambda i: (0, 0, i)),
            pl.BlockSpec((2, 1, B), lambda i: (0, 0, i)),
            pl.BlockSpec((D, 16), lambda i: (0, 0)),
            pl.BlockSpec((2 * D, D), lambda i: (0, 0)),
            pl.BlockSpec((OPAD, D), lambda i: (0, 0)),
        ],
        out_specs=pl.BlockSpec((OPAD, B), lambda i: (0, i)),
        out_shape=jax.ShapeDtypeStruct((OPAD, BSZ), jnp.float32),
        interpret=interpret,
    )(x3, q3, embT, wstack, wlp)


@jax.jit
def kernel(x, x_img, q, embedding, W_linear, W_rule):
    del x_img
    B = 512
    x3 = x.astype(jnp.int32).T[:, None, :]
    q3 = q.astype(jnp.int32).T[:, None, :]
    embT = jnp.zeros((D, 16), jnp.float32).at[:, : NC + 1].set(embedding.T)
    wstack = W_rule.reshape(2 * D, D)
    wlp = jnp.zeros((OPAD, D), jnp.float32).at[:NC].set(W_linear)
    out = _run(x3, q3, embT, wstack, wlp, B=B)
    return out.T[:, :NC]


# speedup vs baseline: 18.9009x; 1.0273x over previous
import functools

import jax
import jax.numpy as jnp
from jax.experimental import pallas as pl

BSZ = 1024
L = 64
D = 32
NC = 10
LVLS = 6
OPAD = 16

_HI = jax.lax.Precision.HIGHEST


def _rms(x):
    return x / (jnp.sqrt(jnp.mean(x * x, axis=1, keepdims=True) + 1e-06) + 1e-06)


def _combine_level(a1, a2, wstack):
    n = a1.shape[0]
    ts = jnp.stack(
        [
            jnp.dot(wstack, a2[j], preferred_element_type=jnp.float32, precision=_HI)
            for j in range(n)
        ],
        axis=0,
    )
    s0 = jnp.sum(a1 * ts[:, :D], axis=1, keepdims=True)
    s1 = jnp.sum(a1 * ts[:, D:], axis=1, keepdims=True)
    m = jnp.maximum(s0, s1)
    e0 = jnp.exp(s0 - m)
    e1 = jnp.exp(s1 - m)
    s = e0 + e1
    return _rms((e0 / s) * a1 + (e1 / s) * a2)


def _tree_body(x3_ref, q3_ref, embT_ref, wstack_ref, wl_ref, out_ref, *, B):
    x3 = x3_ref[...]
    embT = embT_ref[...]
    wstack = wstack_ref[...]
    wl = wl_ref[...]
    ql = q3_ref[0:1]
    qh = q3_ref[1:2]

    h = jnp.zeros((L, D, B), jnp.float32)
    for c in range(NC + 1):
        row = jnp.broadcast_to(embT[:, c : c + 1], (D, B))[None]
        h = jnp.where(x3 == c, row, h)
    h = _rms(h)

    levels = [h]
    for _ in range(LVLS):
        cur = levels[-1]
        n = cur.shape[0]
        pairs = cur.reshape(n // 2, 2, D, B)
        levels.append(_combine_level(pairs[:, 0], pairs[:, 1], wstack))
    levels = levels[::-1]

    inf3 = jnp.broadcast_to(embT[:, NC : NC + 1], (D, B))[None]

    idx = jax.lax.broadcasted_iota(jnp.int32, (L, 1, B), 0)
    full = jnp.logical_and(ql <= idx, qh >= idx)
    res = jnp.where(full, levels[LVLS], inf3)
    n, w = L, 1
    for d in range(LVLS - 1, -1, -1):
        n //= 2
        w *= 2
        pairs = res.reshape(n, 2, D, B)
        comb = _combine_level(pairs[:, 0], pairs[:, 1], wstack)
        j = jax.lax.broadcasted_iota(jnp.int32, (n, 1, B), 0)
        lo = j * w
        hi = lo + (w - 1)
        fullm = jnp.logical_and(ql <= lo, qh >= hi)
        nonem = jnp.logical_or(ql > hi, qh < lo)
        res = jnp.where(fullm, levels[d], jnp.where(nonem, inf3, comb))

    root = res[0]
    out_ref[...] = jnp.dot(wl, root, preferred_element_type=jnp.float32, precision=_HI)


def _run(x3, q3, embT, wstack, wlp, *, B, interpret=False):
    grid = (BSZ // B,)
    return pl.pallas_call(
        functools.partial(_tree_body, B=B),
        grid=grid,
        in_specs=[
            pl.BlockSpec((L, 1, B), lambda i: (0, 0, i)),
            pl.BlockSpec((2, 1, B), lambda i: (0, 0, i)),
            pl.BlockSpec((D, 16), lambda i: (0, 0)),
            pl.BlockSpec((2 * D, D), lambda i: (0, 0)),
            pl.BlockSpec((OPAD, D), lambda i: (0, 0)),
        ],
        out_specs=pl.BlockSpec((OPAD, B), lambda i: (0, i)),
        out_shape=jax.ShapeDtypeStruct((OPAD, BSZ), jnp.float32),
        interpret=interpret,
    )(x3, q3, embT, wstack, wlp)


@jax.jit
def kernel(x, x_img, q, embedding, W_linear, W_rule):
    del x_img
    B = 1024
    x3 = x.astype(jnp.int32).T[:, None, :]
    q3 = q.astype(jnp.int32).T[:, None, :]
    embT = jnp.zeros((D, 16), jnp.float32).at[:, : NC + 1].set(embedding.T)
    wstack = W_rule.reshape(2 * D, D)
    wlp = jnp.zeros((OPAD, D), jnp.float32).at[:NC].set(W_linear)
    out = _run(x3, q3, embT, wstack, wlp, B=B)
    return out.T[:, :NC]
